# Initial kernel scaffold; baseline (speedup 1.0000x reference)
import functools
import math

import jax
import jax.numpy as jnp
from jax import lax
from jax.experimental import pallas as pl
from jax.experimental.pallas import tpu as pltpu
from jax.experimental.pallas import tpu_sc as plsc

NC = 2
NS = 16
L = 16
AUX = 16
CHUNK = 80


def _sc_aggregate(x, src, dst, attr):
    n, d = x.shape
    e = src.shape[0]
    nw = NC * NS
    assert e % (nw * CHUNK) == 0
    assert n % NS == 0
    e_per_w = e // nw
    n_chunks = e_per_w // CHUNK
    rps = n // NS

    zrows = jnp.zeros((rps, d), jnp.float32)
    zaux = jnp.zeros((rps, AUX), jnp.float32)

    mesh = plsc.VectorSubcoreMesh(core_axis_name="c", subcore_axis_name="s")

    @functools.partial(
        pl.kernel,
        out_type=[
            jax.ShapeDtypeStruct((NC * n, d), jnp.float32),
            jax.ShapeDtypeStruct((NC * n, AUX), jnp.float32),
        ],
        mesh=mesh,
        scratch_types=[
            pltpu.VMEM((CHUNK,), jnp.int32),
            pltpu.VMEM((CHUNK,), jnp.int32),
            pltpu.VMEM((CHUNK,), jnp.float32),
            pltpu.VMEM((CHUNK, d), jnp.float32),
            pltpu.VMEM((CHUNK, AUX), jnp.float32),
            pltpu.VMEM_SHARED((n, d), jnp.float32),
            pltpu.VMEM_SHARED((n, AUX), jnp.float32),
            pltpu.SemaphoreType.DMA,
        ],
    )
    def agg(x_hbm, src_hbm, dst_hbm, attr_hbm, zr_hbm, za_hbm,
            pr_hbm, pa_hbm, sidx, didx, attrv, rows, aux, srows, saux, sem):
        cid = lax.axis_index("c")
        sid = lax.axis_index("s")
        wid = sid * NC + cid

        pltpu.sync_copy(zr_hbm, srows.at[pl.ds(sid * rps, rps)])
        pltpu.sync_copy(za_hbm, saux.at[pl.ds(sid * rps, rps)])

        pltpu.sync_copy(za_hbm.at[pl.ds(0, CHUNK)], aux)
        for g in range(CHUNK // L):
            ridx = lax.iota(jnp.int32, L) + g * L
            plsc.store_scatter(aux, [ridx, jnp.ones((L,), jnp.int32)],
                               jnp.ones((L,), jnp.float32))
        plsc.subcore_barrier()

        base0 = wid * e_per_w

        @pl.loop(0, n_chunks)
        def _(i):
            base = base0 + i * CHUNK
            pltpu.sync_copy(src_hbm.at[pl.ds(base, CHUNK)], sidx)
            pltpu.sync_copy(dst_hbm.at[pl.ds(base, CHUNK)], didx)
            pltpu.sync_copy(attr_hbm.at[pl.ds(base, CHUNK)], attrv)
            pltpu.async_copy(x_hbm.at[sidx], rows, sem).wait()
            for g in range(CHUNK // L):
                vals = attrv[pl.ds(g * L, L)]
                ridx = lax.iota(jnp.int32, L) + g * L
                plsc.store_scatter(aux, [ridx, jnp.zeros((L,), jnp.int32)],
                                   vals)
            pltpu.sync_copy(rows, srows.at[didx], add=True)
            pltpu.sync_copy(aux, saux.at[didx], add=True)

        plsc.subcore_barrier()
        out_base = cid * n + sid * rps
        pltpu.sync_copy(srows.at[pl.ds(sid * rps, rps)],
                        pr_hbm.at[pl.ds(out_base, rps)])
        pltpu.sync_copy(saux.at[pl.ds(sid * rps, rps)],
                        pa_hbm.at[pl.ds(out_base, rps)])

    return agg(x, src, dst, attr, zrows, zaux)


def _dense_body(x_ref, pr0_ref, pr1_ref, pa0_ref, pa1_ref, we_ref, be_ref,
                w1_ref, b1_ref, w2_ref, b2_ref, eps_ref, wr_ref, br_ref,
                gamma_ref, beta_ref, out_ref):
    ps = pr0_ref[...] + pr1_ref[...]
    pa = pa0_ref[...] + pa1_ref[...]
    asum = pa[:, 0:1]
    cnt = pa[:, 1:2]
    sums = ps + asum * we_ref[...] + cnt * be_ref[...]
    aggr = jnp.where(cnt > 0.0, sums / jnp.maximum(cnt, 1.0), 0.0)
    xb = x_ref[...]
    h = (1.0 + eps_ref[0, 0]) * xb + aggr
    hp = jax.lax.Precision.HIGHEST
    t = jnp.maximum(
        jnp.dot(h, w1_ref[...], preferred_element_type=jnp.float32,
                precision=hp) + b1_ref[...], 0.0)
    o = jnp.dot(t, w2_ref[...], preferred_element_type=jnp.float32,
                precision=hp) + b2_ref[...]
    o = o + jnp.dot(xb, wr_ref[...], preferred_element_type=jnp.float32,
                    precision=hp) + br_ref[...]
    o = o * (gamma_ref[...] * (1.0 / math.sqrt(1.0 + 1e-5))) + beta_ref[...]
    out_ref[...] = jnp.maximum(o, 0.0)


def _dense(x, pr, pa, We, be, W1, b1, W2, b2, eps, Wr, br, gamma, beta,
           interpret=False):
    n, d = x.shape
    blk = 1000
    assert n % blk == 0
    nb = n // blk

    def row_spec(i_off, w):
        return pl.BlockSpec((blk, w), lambda i, o=i_off: (i + o, 0))

    full = lambda s: pl.BlockSpec(s, lambda i: (0,) * len(s))

    return pl.pallas_call(
        _dense_body,
        grid=(nb,),
        in_specs=[
            row_spec(0, d),
            row_spec(0, d),
            row_spec(nb, d),
            row_spec(0, AUX),
            row_spec(nb, AUX),
            full((1, d)),
            full((1, d)),
            full((d, d)),
            full((1, d)),
            full((d, d)),
            full((1, d)),
            full((1, 1)),
            full((d, d)),
            full((1, d)),
            full((1, d)),
            full((1, d)),
        ],
        out_specs=pl.BlockSpec((blk, d), lambda i: (i, 0)),
        out_shape=jax.ShapeDtypeStruct((n, d), jnp.float32),
        interpret=interpret,
    )(x, pr, pr, pa, pa, We, be.reshape(1, d), W1, b1.reshape(1, d), W2,
      b2.reshape(1, d), eps.reshape(1, 1), Wr, br.reshape(1, d),
      gamma.reshape(1, d), beta.reshape(1, d))


def kernel(x, edge_index, edge_attr, We, be, W1, b1, W2, b2, eps, Wr, br,
           gamma, beta):
    src = edge_index[0]
    dst = edge_index[1]
    attr = edge_attr[:, 0]
    pr, pa = _sc_aggregate(x, src, dst, attr)
    return _dense(x, pr, pa, We, be, W1, b1, W2, b2, eps, Wr, br, gamma,
                  beta)


# speedup vs baseline: 5.1723x; 5.1723x over previous
import dataclasses
import functools
import math

import jax
import jax.numpy as jnp
from jax import lax
from jax.experimental import pallas as pl
from jax.experimental.pallas import tpu as pltpu
from jax.experimental.pallas import tpu_sc as plsc

NC = 2
NS = 16
L = 16
CHUNK = 80


def _sc_aggregate(x, src, dst, attr):
    n, d = x.shape
    e = src.shape[0]
    nw = NC * NS
    assert e % (nw * CHUNK) == 0
    e_per_w = e // nw
    n_chunks = e_per_w // CHUNK
    rps = -(-n // (NS * 128)) * 128
    n_pad = rps * NS
    ar = n_pad // 128

    zrows = jnp.zeros((rps, d), jnp.float32)
    zacc = jnp.zeros((ar, 128), jnp.float32)

    mesh = plsc.VectorSubcoreMesh(core_axis_name="c", subcore_axis_name="s")

    cp = pltpu.CompilerParams()
    if "needs_layout_passes" in pltpu.CompilerParams.__dataclass_fields__:
        cp = dataclasses.replace(cp, needs_layout_passes=False)

    @functools.partial(
        pl.kernel,
        compiler_params=cp,
        out_type=[
            jax.ShapeDtypeStruct((NC, n_pad, d), jnp.float32),
            jax.ShapeDtypeStruct((nw, ar, 128), jnp.float32),
            jax.ShapeDtypeStruct((nw, ar, 128), jnp.float32),
        ],
        mesh=mesh,
        scratch_types=[
            pltpu.VMEM((CHUNK,), jnp.int32),
            pltpu.VMEM((CHUNK,), jnp.int32),
            pltpu.VMEM((CHUNK,), jnp.float32),
            pltpu.VMEM((CHUNK, d), jnp.float32),
            pltpu.VMEM((ar, 128), jnp.float32),
            pltpu.VMEM((ar, 128), jnp.float32),
            pltpu.VMEM_SHARED((n_pad, d), jnp.float32),
            pltpu.SemaphoreType.DMA,
        ],
    )
    def agg(x_hbm, src_hbm, dst_hbm, attr_hbm, zr_hbm, za_hbm,
            pr_hbm, oa_hbm, oc_hbm,
            sidx, didx, attrv, rows, asum, acnt, srows, sem):
        cid = lax.axis_index("c")
        sid = lax.axis_index("s")
        wid = sid * NC + cid

        pltpu.sync_copy(zr_hbm, srows.at[pl.ds(sid * rps, rps)])
        pltpu.sync_copy(za_hbm, asum)
        pltpu.sync_copy(za_hbm, acnt)
        plsc.subcore_barrier()

        base0 = wid * e_per_w
        ones = jnp.ones((L,), jnp.float32)

        @pl.loop(0, n_chunks)
        def _(i):
            base = base0 + i * CHUNK
            pltpu.sync_copy(dst_hbm.at[pl.ds(base, CHUNK)], didx)
            pltpu.sync_copy(attr_hbm.at[pl.ds(base, CHUNK)], attrv)
            for g in range(CHUNK // L):
                dvec = didx[pl.ds(g * L, L)]
                avec = attrv[pl.ds(g * L, L)]
                plsc.addupdate_scatter(asum, [dvec >> 7, dvec & 127], avec)
                plsc.addupdate_scatter(acnt, [dvec >> 7, dvec & 127], ones)
            pltpu.sync_copy(src_hbm.at[pl.ds(base, CHUNK)], sidx)
            pltpu.async_copy(x_hbm.at[sidx], rows, sem).wait()
            pltpu.sync_copy(rows, srows.at[didx], add=True)

        plsc.subcore_barrier()
        pltpu.sync_copy(srows.at[pl.ds(sid * rps, rps)],
                        pr_hbm.at[cid, pl.ds(sid * rps, rps)])
        pltpu.sync_copy(asum, oa_hbm.at[wid])
        pltpu.sync_copy(acnt, oc_hbm.at[wid])

    return agg(x, src, dst, attr, zrows, zacc)


def _dense_body(x_ref, pr_ref, oa_ref, oc_ref, we_ref, be_ref,
                w1_ref, b1_ref, w2_ref, b2_ref, eps_ref, wr_ref, br_ref,
                gamma_ref, beta_ref, out_ref):
    hp = jax.lax.Precision.HIGHEST
    nw = oa_ref.shape[0]
    ones_w = jnp.ones((nw, 1), jnp.float32)
    contract0 = (((0,), (0,)), ((), ()))
    asum = jax.lax.dot_general(oa_ref[...], ones_w, contract0,
                               precision=hp,
                               preferred_element_type=jnp.float32)
    cnt = jax.lax.dot_general(oc_ref[...], ones_w, contract0,
                              precision=hp,
                              preferred_element_type=jnp.float32)
    ps = pr_ref[0] + pr_ref[1]
    sums = ps + asum * we_ref[...] + cnt * be_ref[...]
    aggr = jnp.where(cnt > 0.0, sums / jnp.maximum(cnt, 1.0), 0.0)
    xb = x_ref[...]
    h = (1.0 + eps_ref[0, 0]) * xb + aggr
    t = jnp.maximum(
        jnp.dot(h, w1_ref[...], preferred_element_type=jnp.float32,
                precision=hp) + b1_ref[...], 0.0)
    o = jnp.dot(t, w2_ref[...], preferred_element_type=jnp.float32,
                precision=hp) + b2_ref[...]
    o = o + jnp.dot(xb, wr_ref[...], preferred_element_type=jnp.float32,
                    precision=hp) + br_ref[...]
    o = o * (gamma_ref[...] * (1.0 / math.sqrt(1.0 + 1e-5))) + beta_ref[...]
    out_ref[...] = jnp.maximum(o, 0.0)


def _dense(x, pr, oa, oc, We, be, W1, b1, W2, b2, eps, Wr, br, gamma, beta,
           interpret=False):
    n, d = x.shape
    nc, n_pad, _ = pr.shape
    nw = oa.shape[0]
    oa2 = oa.reshape(nw, n_pad)
    oc2 = oc.reshape(nw, n_pad)
    blk = 1024
    nb = -(-n // blk)
    assert n_pad >= nb * blk

    full = lambda s: pl.BlockSpec(s, lambda i: (0,) * len(s))

    return pl.pallas_call(
        _dense_body,
        grid=(nb,),
        in_specs=[
            pl.BlockSpec((blk, d), lambda i: (i, 0)),
            pl.BlockSpec((nc, blk, d), lambda i: (0, i, 0)),
            pl.BlockSpec((nw, blk), lambda i: (0, i)),
            pl.BlockSpec((nw, blk), lambda i: (0, i)),
            full((1, d)),
            full((1, d)),
            full((d, d)),
            full((1, d)),
            full((d, d)),
            full((1, d)),
            full((1, 1)),
            full((d, d)),
            full((1, d)),
            full((1, d)),
            full((1, d)),
        ],
        out_specs=pl.BlockSpec((blk, d), lambda i: (i, 0)),
        out_shape=jax.ShapeDtypeStruct((n, d), jnp.float32),
        interpret=interpret,
    )(x, pr, oa2, oc2, We, be.reshape(1, d), W1, b1.reshape(1, d), W2,
      b2.reshape(1, d), eps.reshape(1, 1), Wr, br.reshape(1, d),
      gamma.reshape(1, d), beta.reshape(1, d))


def kernel(x, edge_index, edge_attr, We, be, W1, b1, W2, b2, eps, Wr, br,
           gamma, beta):
    src = edge_index[0]
    dst = edge_index[1]
    attr = edge_attr[:, 0]
    pr, oa, oc = _sc_aggregate(x, src, dst, attr)
    return _dense(x, pr, oa, oc, We, be, W1, b1, W2, b2, eps, Wr, br,
                  gamma, beta)


# speedup vs baseline: 11.1064x; 2.1473x over previous
import dataclasses
import functools
import math

import jax
import jax.numpy as jnp
from jax import lax
from jax.experimental import pallas as pl
from jax.experimental.pallas import tpu as pltpu
from jax.experimental.pallas import tpu_sc as plsc

NC = 2
NS = 16
L = 16
CHUNK = 80


def _sc_aggregate(x, src, dst, attr):
    n, d = x.shape
    e = src.shape[0]
    nw = NC * NS
    assert e % (nw * CHUNK) == 0
    e_per_w = e // nw
    n_chunks = e_per_w // CHUNK
    rps = -(-n // (NS * 128)) * 128
    n_pad = rps * NS
    ar = n_pad // 128

    zrows = jnp.zeros((rps, d), jnp.float32)
    zacc = jnp.zeros((ar, 128), jnp.float32)

    mesh = plsc.VectorSubcoreMesh(core_axis_name="c", subcore_axis_name="s")

    cp = pltpu.CompilerParams()
    if "needs_layout_passes" in pltpu.CompilerParams.__dataclass_fields__:
        cp = dataclasses.replace(cp, needs_layout_passes=False)

    assert n_chunks % 4 == 1 and n_chunks >= 9

    @functools.partial(
        pl.kernel,
        compiler_params=cp,
        out_type=[
            jax.ShapeDtypeStruct((NC, n_pad, d), jnp.float32),
            jax.ShapeDtypeStruct((nw, ar, 128), jnp.float32),
            jax.ShapeDtypeStruct((nw, ar, 128), jnp.float32),
        ],
        mesh=mesh,
        scratch_types=(
            [pltpu.VMEM((CHUNK,), jnp.int32)] * 4
            + [pltpu.VMEM((CHUNK,), jnp.int32)] * 4
            + [pltpu.VMEM((CHUNK,), jnp.float32)] * 4
            + [pltpu.VMEM((CHUNK, d), jnp.float32)] * 2
            + [pltpu.VMEM((ar, 128), jnp.float32)] * 2
            + [pltpu.VMEM_SHARED((n_pad, d), jnp.float32)]
            + [pltpu.SemaphoreType.DMA] * 6
        ),
    )
    def agg(x_hbm, src_hbm, dst_hbm, attr_hbm, zr_hbm, za_hbm,
            pr_hbm, oa_hbm, oc_hbm,
            si0, si1, si2, si3, di0, di1, di2, di3, at0, at1, at2, at3,
            rows0, rows1, asum, acnt, srows,
            semI0, semI1, semI2, semI3, semR0, semR1):
        cid = lax.axis_index("c")
        sid = lax.axis_index("s")
        wid = sid * NC + cid
        base0 = wid * e_per_w

        SI = [si0, si1, si2, si3]
        DI = [di0, di1, di2, di3]
        AT = [at0, at1, at2, at3]
        SEMI = [semI0, semI1, semI2, semI3]
        ROWS = [rows0, rows1]
        SEMR = [semR0, semR1]

        pltpu.sync_copy(zr_hbm, srows.at[pl.ds(sid * rps, rps)])
        pltpu.sync_copy(za_hbm, asum)
        pltpu.sync_copy(za_hbm, acnt)
        plsc.subcore_barrier()

        ones = jnp.ones((L,), jnp.float32)

        def issue_idx(c, k4):
            b = base0 + c * CHUNK
            pltpu.async_copy(src_hbm.at[pl.ds(b, CHUNK)], SI[k4], SEMI[k4])
            pltpu.async_copy(dst_hbm.at[pl.ds(b, CHUNK)], DI[k4], SEMI[k4])
            pltpu.async_copy(attr_hbm.at[pl.ds(b, CHUNK)], AT[k4], SEMI[k4])

        def wait_idx(c, k4):
            b = base0 + c * CHUNK
            pltpu.make_async_copy(src_hbm.at[pl.ds(b, CHUNK)], SI[k4],
                                  SEMI[k4]).wait()
            pltpu.make_async_copy(dst_hbm.at[pl.ds(b, CHUNK)], DI[k4],
                                  SEMI[k4]).wait()
            pltpu.make_async_copy(attr_hbm.at[pl.ds(b, CHUNK)], AT[k4],
                                  SEMI[k4]).wait()

        def issue_gather(k4, k2):
            pltpu.async_copy(x_hbm.at[SI[k4]], ROWS[k2], SEMR[k2])

        def wait_gather(k4, k2):
            pltpu.make_async_copy(x_hbm.at[SI[k4]], ROWS[k2],
                                  SEMR[k2]).wait()

        def aux_work(k4):
            for g in range(CHUNK // L):
                dvec = DI[k4][pl.ds(g * L, L)]
                avec = AT[k4][pl.ds(g * L, L)]
                plsc.addupdate_scatter(asum, [dvec >> 7, dvec & 127], avec)
                plsc.addupdate_scatter(acnt, [dvec >> 7, dvec & 127], ones)

        def scatter(k4, k2):
            pltpu.sync_copy(ROWS[k2], srows.at[DI[k4]], add=True)

        issue_idx(0, 0)
        issue_idx(1, 1)
        wait_idx(0, 0)
        issue_gather(0, 0)
        aux_work(0)
        issue_idx(2, 2)
        wait_idx(1, 1)
        issue_gather(1, 1)
        aux_work(1)
        wait_gather(0, 0)
        scatter(0, 0)
        issue_idx(3, 3)

        @pl.loop(0, (n_chunks - 5) // 4)
        def _(j):
            c_base = 4 * j + 2
            for o in range(4):
                c = c_base + o
                k4 = (2 + o) % 4
                k2 = o % 2
                wait_idx(c, k4)
                issue_gather(k4, k2)
                aux_work(k4)
                wait_gather((k4 - 1) % 4, (k2 + 1) % 2)
                scatter((k4 - 1) % 4, (k2 + 1) % 2)
                issue_idx(c + 2, (k4 + 2) % 4)

        nc3 = n_chunks - 3
        wait_idx(nc3, 2)
        issue_gather(2, 0)
        aux_work(2)
        wait_gather(1, 1)
        scatter(1, 1)
        issue_idx(nc3 + 2, 0)
        wait_idx(nc3 + 1, 3)
        issue_gather(3, 1)
        aux_work(3)
        wait_gather(2, 0)
        scatter(2, 0)
        wait_idx(nc3 + 2, 0)
        issue_gather(0, 0)
        aux_work(0)
        wait_gather(3, 1)
        scatter(3, 1)
        wait_gather(0, 0)
        scatter(0, 0)

        plsc.subcore_barrier()
        pltpu.sync_copy(srows.at[pl.ds(sid * rps, rps)],
                        pr_hbm.at[cid, pl.ds(sid * rps, rps)])
        pltpu.sync_copy(asum, oa_hbm.at[wid])
        pltpu.sync_copy(acnt, oc_hbm.at[wid])

    return agg(x, src, dst, attr, zrows, zacc)


def _dense_body(x_ref, pr_ref, oa_ref, oc_ref, we_ref, be_ref,
                w1_ref, b1_ref, w2_ref, b2_ref, eps_ref, wr_ref, br_ref,
                gamma_ref, beta_ref, out_ref):
    hp = jax.lax.Precision.HIGHEST
    nw = oa_ref.shape[0]
    ones_w = jnp.ones((nw, 1), jnp.float32)
    contract0 = (((0,), (0,)), ((), ()))
    asum = jax.lax.dot_general(oa_ref[...], ones_w, contract0,
                               precision=hp,
                               preferred_element_type=jnp.float32)
    cnt = jax.lax.dot_general(oc_ref[...], ones_w, contract0,
                              precision=hp,
                              preferred_element_type=jnp.float32)
    ps = pr_ref[0] + pr_ref[1]
    sums = ps + asum * we_ref[...] + cnt * be_ref[...]
    aggr = jnp.where(cnt > 0.0, sums / jnp.maximum(cnt, 1.0), 0.0)
    xb = x_ref[...]
    h = (1.0 + eps_ref[0, 0]) * xb + aggr
    t = jnp.maximum(
        jnp.dot(h, w1_ref[...], preferred_element_type=jnp.float32,
                precision=hp) + b1_ref[...], 0.0)
    o = jnp.dot(t, w2_ref[...], preferred_element_type=jnp.float32,
                precision=hp) + b2_ref[...]
    o = o + jnp.dot(xb, wr_ref[...], preferred_element_type=jnp.float32,
                    precision=hp) + br_ref[...]
    o = o * (gamma_ref[...] * (1.0 / math.sqrt(1.0 + 1e-5))) + beta_ref[...]
    out_ref[...] = jnp.maximum(o, 0.0)


def _dense(x, pr, oa, oc, We, be, W1, b1, W2, b2, eps, Wr, br, gamma, beta,
           interpret=False):
    n, d = x.shape
    nc, n_pad, _ = pr.shape
    nw = oa.shape[0]
    oa2 = oa.reshape(nw, n_pad)
    oc2 = oc.reshape(nw, n_pad)
    blk = 1024
    nb = -(-n // blk)
    assert n_pad >= nb * blk

    full = lambda s: pl.BlockSpec(s, lambda i: (0,) * len(s))

    return pl.pallas_call(
        _dense_body,
        grid=(nb,),
        in_specs=[
            pl.BlockSpec((blk, d), lambda i: (i, 0)),
            pl.BlockSpec((nc, blk, d), lambda i: (0, i, 0)),
            pl.BlockSpec((nw, blk), lambda i: (0, i)),
            pl.BlockSpec((nw, blk), lambda i: (0, i)),
            full((1, d)),
            full((1, d)),
            full((d, d)),
            full((1, d)),
            full((d, d)),
            full((1, d)),
            full((1, 1)),
            full((d, d)),
            full((1, d)),
            full((1, d)),
            full((1, d)),
        ],
        out_specs=pl.BlockSpec((blk, d), lambda i: (i, 0)),
        out_shape=jax.ShapeDtypeStruct((n, d), jnp.float32),
        interpret=interpret,
    )(x, pr, oa2, oc2, We, be.reshape(1, d), W1, b1.reshape(1, d), W2,
      b2.reshape(1, d), eps.reshape(1, 1), Wr, br.reshape(1, d),
      gamma.reshape(1, d), beta.reshape(1, d))


def kernel(x, edge_index, edge_attr, We, be, W1, b1, W2, b2, eps, Wr, br,
           gamma, beta):
    src = edge_index[0]
    dst = edge_index[1]
    attr = edge_attr[:, 0]
    pr, oa, oc = _sc_aggregate(x, src, dst, attr)
    return _dense(x, pr, oa, oc, We, be, W1, b1, W2, b2, eps, Wr, br,
                  gamma, beta)


# speedup vs baseline: 11.1420x; 1.0032x over previous
import dataclasses
import functools
import math

import jax
import jax.numpy as jnp
from jax import lax
from jax.experimental import pallas as pl
from jax.experimental.pallas import tpu as pltpu
from jax.experimental.pallas import tpu_sc as plsc

NC = 2
NS = 16
L = 16
CHUNK = 80


def _sc_aggregate(x, src, dst, attr):
    n, d = x.shape
    e = src.shape[0]
    nw = NC * NS
    assert e % (nw * CHUNK) == 0
    e_per_w = e // nw
    n_chunks = e_per_w // CHUNK
    rps = -(-n // (NS * 128)) * 128
    n_pad = rps * NS
    ar = n_pad // 128

    zrows = jnp.zeros((rps, d), jnp.float32)
    zacc = jnp.zeros((ar, 128), jnp.float32)

    mesh = plsc.VectorSubcoreMesh(core_axis_name="c", subcore_axis_name="s")

    cp = pltpu.CompilerParams()
    if "needs_layout_passes" in pltpu.CompilerParams.__dataclass_fields__:
        cp = dataclasses.replace(cp, needs_layout_passes=False)

    assert n_chunks % 4 == 1 and n_chunks >= 9

    @functools.partial(
        pl.kernel,
        compiler_params=cp,
        out_type=[
            jax.ShapeDtypeStruct((NC, n_pad, d), jnp.float32),
            jax.ShapeDtypeStruct((nw, ar, 128), jnp.float32),
            jax.ShapeDtypeStruct((nw, ar, 128), jnp.float32),
        ],
        mesh=mesh,
        scratch_types=(
            [pltpu.VMEM((CHUNK,), jnp.int32)] * 4
            + [pltpu.VMEM((CHUNK,), jnp.int32)] * 4
            + [pltpu.VMEM((CHUNK,), jnp.float32)] * 4
            + [pltpu.VMEM((CHUNK, d), jnp.float32)] * 2
            + [pltpu.VMEM((ar, 128), jnp.float32)] * 2
            + [pltpu.VMEM_SHARED((n_pad, d), jnp.float32)]
            + [pltpu.SemaphoreType.DMA] * 8
        ),
    )
    def agg(x_hbm, src_hbm, dst_hbm, attr_hbm, zr_hbm, za_hbm,
            pr_hbm, oa_hbm, oc_hbm,
            si0, si1, si2, si3, di0, di1, di2, di3, at0, at1, at2, at3,
            rows0, rows1, asum, acnt, srows,
            semI0, semI1, semI2, semI3, semR0, semR1, semS0, semS1):
        cid = lax.axis_index("c")
        sid = lax.axis_index("s")
        wid = sid * NC + cid
        base0 = wid * e_per_w

        SI = [si0, si1, si2, si3]
        DI = [di0, di1, di2, di3]
        AT = [at0, at1, at2, at3]
        SEMI = [semI0, semI1, semI2, semI3]
        ROWS = [rows0, rows1]
        SEMR = [semR0, semR1]
        SEMS = [semS0, semS1]

        ones = jnp.ones((L,), jnp.float32)

        def issue_idx(c, k4):
            b = base0 + c * CHUNK
            pltpu.async_copy(src_hbm.at[pl.ds(b, CHUNK)], SI[k4], SEMI[k4])
            pltpu.async_copy(dst_hbm.at[pl.ds(b, CHUNK)], DI[k4], SEMI[k4])
            pltpu.async_copy(attr_hbm.at[pl.ds(b, CHUNK)], AT[k4], SEMI[k4])

        def wait_idx(c, k4):
            b = base0 + c * CHUNK
            pltpu.make_async_copy(src_hbm.at[pl.ds(b, CHUNK)], SI[k4],
                                  SEMI[k4]).wait()
            pltpu.make_async_copy(dst_hbm.at[pl.ds(b, CHUNK)], DI[k4],
                                  SEMI[k4]).wait()
            pltpu.make_async_copy(attr_hbm.at[pl.ds(b, CHUNK)], AT[k4],
                                  SEMI[k4]).wait()

        def issue_gather(k4, k2):
            pltpu.async_copy(x_hbm.at[SI[k4]], ROWS[k2], SEMR[k2])

        def wait_gather(k4, k2):
            pltpu.make_async_copy(x_hbm.at[SI[k4]], ROWS[k2],
                                  SEMR[k2]).wait()

        def aux_work(k4):
            for g in range(CHUNK // L):
                dvec = DI[k4][pl.ds(g * L, L)]
                avec = AT[k4][pl.ds(g * L, L)]
                plsc.addupdate_scatter(asum, [dvec >> 7, dvec & 127], avec)
                plsc.addupdate_scatter(acnt, [dvec >> 7, dvec & 127], ones)

        def scatter(k4, k2):
            pltpu.async_copy(ROWS[k2], srows.at[DI[k4]], SEMS[k2],
                             add=True)

        def wait_scatter(k4, k2):
            pltpu.make_async_copy(ROWS[k2], srows.at[DI[k4]],
                                  SEMS[k2]).wait()

        issue_idx(0, 0)
        issue_idx(1, 1)
        issue_idx(2, 2)
        issue_idx(3, 3)
        pltpu.sync_copy(zr_hbm, srows.at[pl.ds(sid * rps, rps)])
        pltpu.sync_copy(za_hbm, asum)
        pltpu.sync_copy(za_hbm, acnt)
        plsc.subcore_barrier()
        wait_idx(0, 0)
        issue_gather(0, 0)
        aux_work(0)
        wait_idx(1, 1)
        issue_gather(1, 1)
        aux_work(1)
        wait_gather(0, 0)
        scatter(0, 0)

        @pl.loop(0, (n_chunks - 5) // 4)
        def _(j):
            c_base = 4 * j + 2
            for o in range(4):
                c = c_base + o
                k4 = (2 + o) % 4
                k2 = o % 2
                wait_idx(c, k4)
                wait_scatter((k4 - 2) % 4, k2)
                issue_gather(k4, k2)
                aux_work(k4)
                wait_gather((k4 - 1) % 4, (k2 + 1) % 2)
                scatter((k4 - 1) % 4, (k2 + 1) % 2)
                issue_idx(c + 2, (k4 + 2) % 4)

        nc3 = n_chunks - 3
        wait_idx(nc3, 2)
        wait_scatter(0, 0)
        issue_gather(2, 0)
        aux_work(2)
        wait_gather(1, 1)
        scatter(1, 1)
        issue_idx(nc3 + 2, 0)
        wait_idx(nc3 + 1, 3)
        wait_scatter(1, 1)
        issue_gather(3, 1)
        aux_work(3)
        wait_gather(2, 0)
        scatter(2, 0)
        wait_idx(nc3 + 2, 0)
        wait_scatter(2, 0)
        issue_gather(0, 0)
        aux_work(0)
        wait_gather(3, 1)
        scatter(3, 1)
        wait_gather(0, 0)
        scatter(0, 0)
        wait_scatter(3, 1)
        wait_scatter(0, 0)

        plsc.subcore_barrier()
        pltpu.sync_copy(srows.at[pl.ds(sid * rps, rps)],
                        pr_hbm.at[cid, pl.ds(sid * rps, rps)])
        pltpu.sync_copy(asum, oa_hbm.at[wid])
        pltpu.sync_copy(acnt, oc_hbm.at[wid])

    return agg(x, src, dst, attr, zrows, zacc)


def _dense_body(x_ref, pr_ref, oa_ref, oc_ref, we_ref, be_ref,
                w1_ref, b1_ref, w2_ref, b2_ref, eps_ref, wr_ref, br_ref,
                gamma_ref, beta_ref, out_ref):
    hp = jax.lax.Precision.HIGHEST
    nw = oa_ref.shape[0]
    ones_w = jnp.ones((nw, 1), jnp.float32)
    contract0 = (((0,), (0,)), ((), ()))
    asum = jax.lax.dot_general(oa_ref[...], ones_w, contract0,
                               precision=hp,
                               preferred_element_type=jnp.float32)
    cnt = jax.lax.dot_general(oc_ref[...], ones_w, contract0,
                              precision=hp,
                              preferred_element_type=jnp.float32)
    ps = pr_ref[0] + pr_ref[1]
    sums = ps + asum * we_ref[...] + cnt * be_ref[...]
    aggr = jnp.where(cnt > 0.0, sums / jnp.maximum(cnt, 1.0), 0.0)
    xb = x_ref[...]
    h = (1.0 + eps_ref[0, 0]) * xb + aggr
    t = jnp.maximum(
        jnp.dot(h, w1_ref[...], preferred_element_type=jnp.float32,
                precision=hp) + b1_ref[...], 0.0)
    o = jnp.dot(t, w2_ref[...], preferred_element_type=jnp.float32,
                precision=hp) + b2_ref[...]
    o = o + jnp.dot(xb, wr_ref[...], preferred_element_type=jnp.float32,
                    precision=hp) + br_ref[...]
    o = o * (gamma_ref[...] * (1.0 / math.sqrt(1.0 + 1e-5))) + beta_ref[...]
    out_ref[...] = jnp.maximum(o, 0.0)


def _dense(x, pr, oa, oc, We, be, W1, b1, W2, b2, eps, Wr, br, gamma, beta,
           interpret=False):
    n, d = x.shape
    nc, n_pad, _ = pr.shape
    nw = oa.shape[0]
    oa2 = oa.reshape(nw, n_pad)
    oc2 = oc.reshape(nw, n_pad)
    blk = 1024
    nb = -(-n // blk)
    assert n_pad >= nb * blk

    full = lambda s: pl.BlockSpec(s, lambda i: (0,) * len(s))

    return pl.pallas_call(
        _dense_body,
        grid=(nb,),
        in_specs=[
            pl.BlockSpec((blk, d), lambda i: (i, 0)),
            pl.BlockSpec((nc, blk, d), lambda i: (0, i, 0)),
            pl.BlockSpec((nw, blk), lambda i: (0, i)),
            pl.BlockSpec((nw, blk), lambda i: (0, i)),
            full((1, d)),
            full((1, d)),
            full((d, d)),
            full((1, d)),
            full((d, d)),
            full((1, d)),
            full((1, 1)),
            full((d, d)),
            full((1, d)),
            full((1, d)),
            full((1, d)),
        ],
        out_specs=pl.BlockSpec((blk, d), lambda i: (i, 0)),
        out_shape=jax.ShapeDtypeStruct((n, d), jnp.float32),
        interpret=interpret,
    )(x, pr, oa2, oc2, We, be.reshape(1, d), W1, b1.reshape(1, d), W2,
      b2.reshape(1, d), eps.reshape(1, 1), Wr, br.reshape(1, d),
      gamma.reshape(1, d), beta.reshape(1, d))


def kernel(x, edge_index, edge_attr, We, be, W1, b1, W2, b2, eps, Wr, br,
           gamma, beta):
    src = edge_index[0]
    dst = edge_index[1]
    attr = edge_attr[:, 0]
    pr, oa, oc = _sc_aggregate(x, src, dst, attr)
    return _dense(x, pr, oa, oc, We, be, W1, b1, W2, b2, eps, Wr, br,
                  gamma, beta)


# speedup vs baseline: 11.9167x; 1.0695x over previous
import dataclasses
import functools
import math

import jax
import jax.numpy as jnp
from jax import lax
from jax.experimental import pallas as pl
from jax.experimental.pallas import tpu as pltpu
from jax.experimental.pallas import tpu_sc as plsc

NC = 2
NS = 16
L = 16
CHUNK = 80


def _sc_aggregate(x, src, dst, attr):
    n, d = x.shape
    e = src.shape[0]
    nw = NC * NS
    assert e % (nw * CHUNK) == 0
    e_per_w = e // nw
    n_chunks = e_per_w // CHUNK
    rps = -(-n // (NS * 128)) * 128
    n_pad = rps * NS
    ar = n_pad // 128

    zrows = jnp.zeros((rps, d), jnp.float32)
    zacc = jnp.zeros((ar, 128), jnp.float32)

    mesh = plsc.VectorSubcoreMesh(core_axis_name="c", subcore_axis_name="s")

    cp = pltpu.CompilerParams()
    if "needs_layout_passes" in pltpu.CompilerParams.__dataclass_fields__:
        cp = dataclasses.replace(cp, needs_layout_passes=False)

    assert n_chunks % 4 == 1 and n_chunks >= 9

    @functools.partial(
        pl.kernel,
        compiler_params=cp,
        out_type=[
            jax.ShapeDtypeStruct((NC, n_pad, d), jnp.float32),
            jax.ShapeDtypeStruct((nw, ar, 128), jnp.float32),
            jax.ShapeDtypeStruct((nw, ar, 128), jnp.float32),
        ],
        mesh=mesh,
        scratch_types=(
            [pltpu.VMEM((CHUNK,), jnp.int32)] * 4
            + [pltpu.VMEM((CHUNK,), jnp.int32)] * 4
            + [pltpu.VMEM((CHUNK,), jnp.float32)] * 4
            + [pltpu.VMEM((CHUNK, d), jnp.float32)] * 2
            + [pltpu.VMEM((ar, 128), jnp.float32)] * 2
            + [pltpu.VMEM_SHARED((n_pad, d), jnp.float32)]
            + [pltpu.SemaphoreType.DMA] * 8
        ),
    )
    def agg(x_hbm, src_hbm, dst_hbm, attr_hbm, zr_hbm, za_hbm,
            pr_hbm, oa_hbm, oc_hbm,
            si0, si1, si2, si3, di0, di1, di2, di3, at0, at1, at2, at3,
            rows0, rows1, asum, acnt, srows,
            semI0, semI1, semI2, semI3, semR0, semR1, semS0, semS1):
        cid = lax.axis_index("c")
        sid = lax.axis_index("s")
        wid = sid * NC + cid
        base0 = wid * e_per_w

        SI = [si0, si1, si2, si3]
        DI = [di0, di1, di2, di3]
        AT = [at0, at1, at2, at3]
        SEMI = [semI0, semI1, semI2, semI3]
        ROWS = [rows0, rows1]
        SEMR = [semR0, semR1]
        SEMS = [semS0, semS1]

        ones = jnp.ones((L,), jnp.float32)

        def issue_idx(c, k4):
            b = base0 + c * CHUNK
            pltpu.async_copy(src_hbm.at[pl.ds(b, CHUNK)], SI[k4], SEMI[k4])
            pltpu.async_copy(dst_hbm.at[pl.ds(b, CHUNK)], DI[k4], SEMI[k4])
            pltpu.async_copy(attr_hbm.at[pl.ds(b, CHUNK)], AT[k4], SEMI[k4])

        def wait_idx(c, k4):
            b = base0 + c * CHUNK
            pltpu.make_async_copy(src_hbm.at[pl.ds(b, CHUNK)], SI[k4],
                                  SEMI[k4]).wait()
            pltpu.make_async_copy(dst_hbm.at[pl.ds(b, CHUNK)], DI[k4],
                                  SEMI[k4]).wait()
            pltpu.make_async_copy(attr_hbm.at[pl.ds(b, CHUNK)], AT[k4],
                                  SEMI[k4]).wait()

        def issue_gather(k4, k2):
            pltpu.async_copy(x_hbm.at[SI[k4]], ROWS[k2], SEMR[k2])

        def wait_gather(k4, k2):
            pltpu.make_async_copy(x_hbm.at[SI[k4]], ROWS[k2],
                                  SEMR[k2]).wait()

        def aux_work(k4):
            for g in range(CHUNK // L):
                dvec = DI[k4][pl.ds(g * L, L)]
                avec = AT[k4][pl.ds(g * L, L)]
                plsc.addupdate_scatter(asum, [dvec >> 7, dvec & 127], avec)
                plsc.addupdate_scatter(acnt, [dvec >> 7, dvec & 127], ones)

        def scatter(k4, k2):
            pltpu.async_copy(ROWS[k2], srows.at[DI[k4]], SEMS[k2],
                             add=True)

        def wait_scatter(k4, k2):
            pltpu.make_async_copy(ROWS[k2], srows.at[DI[k4]],
                                  SEMS[k2]).wait()

        issue_idx(0, 0)
        issue_idx(1, 1)
        issue_idx(2, 2)
        issue_idx(3, 3)
        pltpu.sync_copy(zr_hbm, srows.at[pl.ds(sid * rps, rps)])
        pltpu.sync_copy(za_hbm, asum)
        pltpu.sync_copy(za_hbm, acnt)
        plsc.subcore_barrier()
        wait_idx(0, 0)
        issue_gather(0, 0)
        aux_work(0)
        wait_idx(1, 1)
        issue_gather(1, 1)
        aux_work(1)
        wait_gather(0, 0)
        scatter(0, 0)

        @pl.loop(0, (n_chunks - 5) // 4)
        def _(j):
            c_base = 4 * j + 2
            for o in range(4):
                c = c_base + o
                k4 = (2 + o) % 4
                k2 = o % 2
                wait_idx(c, k4)
                wait_scatter((k4 - 2) % 4, k2)
                issue_gather(k4, k2)
                aux_work(k4)
                wait_gather((k4 - 1) % 4, (k2 + 1) % 2)
                scatter((k4 - 1) % 4, (k2 + 1) % 2)
                issue_idx(c + 2, (k4 + 2) % 4)

        nc3 = n_chunks - 3
        wait_idx(nc3, 2)
        wait_scatter(0, 0)
        issue_gather(2, 0)
        aux_work(2)
        wait_gather(1, 1)
        scatter(1, 1)
        issue_idx(nc3 + 2, 0)
        wait_idx(nc3 + 1, 3)
        wait_scatter(1, 1)
        issue_gather(3, 1)
        aux_work(3)
        wait_gather(2, 0)
        scatter(2, 0)
        wait_idx(nc3 + 2, 0)
        wait_scatter(2, 0)
        issue_gather(0, 0)
        aux_work(0)
        wait_gather(3, 1)
        scatter(3, 1)
        wait_gather(0, 0)
        scatter(0, 0)
        wait_scatter(3, 1)
        wait_scatter(0, 0)

        plsc.subcore_barrier()
        pltpu.sync_copy(srows.at[pl.ds(sid * rps, rps)],
                        pr_hbm.at[cid, pl.ds(sid * rps, rps)])
        pltpu.sync_copy(asum, oa_hbm.at[wid])
        pltpu.sync_copy(acnt, oc_hbm.at[wid])

    return agg(x, src, dst, attr, zrows, zacc)


def _mm3(a, b):
    hp = jax.lax.Precision.DEFAULT
    a_hi = a.astype(jnp.bfloat16)
    a_lo = (a - a_hi.astype(jnp.float32)).astype(jnp.bfloat16)
    b_hi = b.astype(jnp.bfloat16)
    b_lo = (b - b_hi.astype(jnp.float32)).astype(jnp.bfloat16)
    f = lambda u, v: jnp.dot(u, v, preferred_element_type=jnp.float32,
                             precision=hp)
    return f(a_hi, b_hi) + (f(a_lo, b_hi) + f(a_hi, b_lo))


def _dense_body(x_ref, pr_ref, oa_ref, oc_ref, we_ref, be_ref,
                w1_ref, b1_ref, w2_ref, b2_ref, eps_ref, wr_ref, br_ref,
                gamma_ref, beta_ref, out_ref):
    hp = jax.lax.Precision.HIGHEST
    nw = oa_ref.shape[0]
    ones_w = jnp.ones((nw, 1), jnp.float32)
    contract0 = (((0,), (0,)), ((), ()))
    asum = jax.lax.dot_general(oa_ref[...], ones_w, contract0,
                               precision=hp,
                               preferred_element_type=jnp.float32)
    cnt = jax.lax.dot_general(oc_ref[...], ones_w, contract0,
                              precision=hp,
                              preferred_element_type=jnp.float32)
    ps = pr_ref[0] + pr_ref[1]
    sums = ps + asum * we_ref[...] + cnt * be_ref[...]
    aggr = jnp.where(cnt > 0.0, sums / jnp.maximum(cnt, 1.0), 0.0)
    xb = x_ref[...]
    h = (1.0 + eps_ref[0, 0]) * xb + aggr
    t = jnp.maximum(_mm3(h, w1_ref[...]) + b1_ref[...], 0.0)
    o = _mm3(t, w2_ref[...]) + b2_ref[...]
    o = o + _mm3(xb, wr_ref[...]) + br_ref[...]
    o = o * (gamma_ref[...] * (1.0 / math.sqrt(1.0 + 1e-5))) + beta_ref[...]
    out_ref[...] = jnp.maximum(o, 0.0)


def _dense(x, pr, oa, oc, We, be, W1, b1, W2, b2, eps, Wr, br, gamma, beta,
           interpret=False):
    n, d = x.shape
    nc, n_pad, _ = pr.shape
    nw = oa.shape[0]
    oa2 = oa.reshape(nw, n_pad)
    oc2 = oc.reshape(nw, n_pad)
    blk = 1024
    nb = -(-n // blk)
    assert n_pad >= nb * blk

    full = lambda s: pl.BlockSpec(s, lambda i: (0,) * len(s))

    return pl.pallas_call(
        _dense_body,
        grid=(nb,),
        in_specs=[
            pl.BlockSpec((blk, d), lambda i: (i, 0)),
            pl.BlockSpec((nc, blk, d), lambda i: (0, i, 0)),
            pl.BlockSpec((nw, blk), lambda i: (0, i)),
            pl.BlockSpec((nw, blk), lambda i: (0, i)),
            full((1, d)),
            full((1, d)),
            full((d, d)),
            full((1, d)),
            full((d, d)),
            full((1, d)),
            full((1, 1)),
            full((d, d)),
            full((1, d)),
            full((1, d)),
            full((1, d)),
        ],
        out_specs=pl.BlockSpec((blk, d), lambda i: (i, 0)),
        out_shape=jax.ShapeDtypeStruct((n, d), jnp.float32),
        interpret=interpret,
    )(x, pr, oa2, oc2, We, be.reshape(1, d), W1, b1.reshape(1, d), W2,
      b2.reshape(1, d), eps.reshape(1, 1), Wr, br.reshape(1, d),
      gamma.reshape(1, d), beta.reshape(1, d))


def kernel(x, edge_index, edge_attr, We, be, W1, b1, W2, b2, eps, Wr, br,
           gamma, beta):
    src = edge_index[0]
    dst = edge_index[1]
    attr = edge_attr[:, 0]
    pr, oa, oc = _sc_aggregate(x, src, dst, attr)
    return _dense(x, pr, oa, oc, We, be, W1, b1, W2, b2, eps, Wr, br,
                  gamma, beta)


# speedup vs baseline: 12.0867x; 1.0143x over previous
import dataclasses
import functools
import math

import jax
import jax.numpy as jnp
from jax import lax
from jax.experimental import pallas as pl
from jax.experimental.pallas import tpu as pltpu
from jax.experimental.pallas import tpu_sc as plsc

NC = 2
NS = 16
L = 16
CHUNK = 80


def _sc_aggregate(x, src, dst, attr):
    n, d = x.shape
    e = src.shape[0]
    nw = NC * NS
    assert e % (nw * CHUNK) == 0
    e_per_w = e // nw
    n_chunks = e_per_w // CHUNK
    rps = -(-n // (NS * 128)) * 128
    n_pad = rps * NS
    ar = n_pad // 128

    zrows = jnp.zeros((rps, d), jnp.float32)
    zacc = jnp.zeros((ar, 128), jnp.float32)

    mesh = plsc.VectorSubcoreMesh(core_axis_name="c", subcore_axis_name="s")

    cp = pltpu.CompilerParams()
    if "needs_layout_passes" in pltpu.CompilerParams.__dataclass_fields__:
        cp = dataclasses.replace(cp, needs_layout_passes=False)

    assert n_chunks % 4 == 1 and n_chunks >= 9

    @functools.partial(
        pl.kernel,
        compiler_params=cp,
        out_type=[
            jax.ShapeDtypeStruct((NC, n_pad, d), jnp.float32),
            jax.ShapeDtypeStruct((nw, ar, 128), jnp.float32),
            jax.ShapeDtypeStruct((nw, ar, 128), jnp.float32),
        ],
        mesh=mesh,
        scratch_types=(
            [pltpu.VMEM((CHUNK,), jnp.int32)] * 4
            + [pltpu.VMEM((CHUNK,), jnp.int32)] * 4
            + [pltpu.VMEM((CHUNK,), jnp.float32)] * 4
            + [pltpu.VMEM((CHUNK, d), jnp.float32)] * 2
            + [pltpu.VMEM((ar, 128), jnp.float32)] * 2
            + [pltpu.VMEM_SHARED((n_pad, d), jnp.float32)]
            + [pltpu.SemaphoreType.DMA] * 8
        ),
    )
    def agg(x_hbm, src_hbm, dst_hbm, attr_hbm, zr_hbm, za_hbm,
            pr_hbm, oa_hbm, oc_hbm,
            si0, si1, si2, si3, di0, di1, di2, di3, at0, at1, at2, at3,
            rows0, rows1, asum, acnt, srows,
            semI0, semI1, semI2, semI3, semR0, semR1, semS0, semS1):
        cid = lax.axis_index("c")
        sid = lax.axis_index("s")
        wid = sid * NC + cid
        base0 = wid * e_per_w

        SI = [si0, si1, si2, si3]
        DI = [di0, di1, di2, di3]
        AT = [at0, at1, at2, at3]
        SEMI = [semI0, semI1, semI2, semI3]
        ROWS = [rows0, rows1]
        SEMR = [semR0, semR1]
        SEMS = [semS0, semS1]

        ones = jnp.ones((L,), jnp.float32)

        def issue_idx(c, k4):
            b = base0 + c * CHUNK
            pltpu.async_copy(src_hbm.at[pl.ds(b, CHUNK)], SI[k4], SEMI[k4])
            pltpu.async_copy(dst_hbm.at[pl.ds(b, CHUNK)], DI[k4], SEMI[k4])
            pltpu.async_copy(attr_hbm.at[pl.ds(b, CHUNK)], AT[k4], SEMI[k4])

        def wait_idx(c, k4):
            b = base0 + c * CHUNK
            pltpu.make_async_copy(src_hbm.at[pl.ds(b, CHUNK)], SI[k4],
                                  SEMI[k4]).wait()
            pltpu.make_async_copy(dst_hbm.at[pl.ds(b, CHUNK)], DI[k4],
                                  SEMI[k4]).wait()
            pltpu.make_async_copy(attr_hbm.at[pl.ds(b, CHUNK)], AT[k4],
                                  SEMI[k4]).wait()

        def issue_gather(k4, k2):
            pltpu.async_copy(x_hbm.at[SI[k4]], ROWS[k2], SEMR[k2])

        def wait_gather(k4, k2):
            pltpu.make_async_copy(x_hbm.at[SI[k4]], ROWS[k2],
                                  SEMR[k2]).wait()

        def aux_work(k4):
            for g in range(CHUNK // L):
                dvec = DI[k4][pl.ds(g * L, L)]
                avec = AT[k4][pl.ds(g * L, L)]
                plsc.addupdate_scatter(asum, [dvec >> 7, dvec & 127], avec)
                plsc.addupdate_scatter(acnt, [dvec >> 7, dvec & 127], ones)

        def scatter(k4, k2):
            pltpu.async_copy(ROWS[k2], srows.at[DI[k4]], SEMS[k2],
                             add=True)

        def wait_scatter(k4, k2):
            pltpu.make_async_copy(ROWS[k2], srows.at[DI[k4]],
                                  SEMS[k2]).wait()

        issue_idx(0, 0)
        issue_idx(1, 1)
        issue_idx(2, 2)
        issue_idx(3, 3)
        pltpu.sync_copy(zr_hbm, srows.at[pl.ds(sid * rps, rps)])
        pltpu.sync_copy(za_hbm, asum)
        pltpu.sync_copy(za_hbm, acnt)
        plsc.subcore_barrier()
        wait_idx(0, 0)
        issue_gather(0, 0)
        aux_work(0)
        wait_idx(1, 1)
        issue_gather(1, 1)
        aux_work(1)
        wait_gather(0, 0)
        scatter(0, 0)

        @pl.loop(0, (n_chunks - 5) // 4)
        def _(j):
            c_base = 4 * j + 2
            for o in range(4):
                c = c_base + o
                k4 = (2 + o) % 4
                k2 = o % 2
                wait_idx(c, k4)
                wait_scatter((k4 - 2) % 4, k2)
                issue_gather(k4, k2)
                aux_work(k4)
                wait_gather((k4 - 1) % 4, (k2 + 1) % 2)
                scatter((k4 - 1) % 4, (k2 + 1) % 2)
                issue_idx(c + 2, (k4 + 2) % 4)

        nc3 = n_chunks - 3
        wait_idx(nc3, 2)
        wait_scatter(0, 0)
        issue_gather(2, 0)
        aux_work(2)
        wait_gather(1, 1)
        scatter(1, 1)
        issue_idx(nc3 + 2, 0)
        wait_idx(nc3 + 1, 3)
        wait_scatter(1, 1)
        issue_gather(3, 1)
        aux_work(3)
        wait_gather(2, 0)
        scatter(2, 0)
        wait_idx(nc3 + 2, 0)
        wait_scatter(2, 0)
        issue_gather(0, 0)
        aux_work(0)
        wait_gather(3, 1)
        scatter(3, 1)
        wait_gather(0, 0)
        scatter(0, 0)
        wait_scatter(3, 1)
        wait_scatter(0, 0)

        pltpu.sync_copy(asum, oa_hbm.at[wid])
        pltpu.sync_copy(acnt, oc_hbm.at[wid])
        plsc.subcore_barrier()
        pltpu.sync_copy(srows.at[pl.ds(sid * rps, rps)],
                        pr_hbm.at[cid, pl.ds(sid * rps, rps)])

    return agg(x, src, dst, attr, zrows, zacc)


def _mm3(a, b):
    hp = jax.lax.Precision.DEFAULT
    a_hi = a.astype(jnp.bfloat16)
    a_lo = (a - a_hi.astype(jnp.float32)).astype(jnp.bfloat16)
    b_hi = b.astype(jnp.bfloat16)
    b_lo = (b - b_hi.astype(jnp.float32)).astype(jnp.bfloat16)
    f = lambda u, v: jnp.dot(u, v, preferred_element_type=jnp.float32,
                             precision=hp)
    return f(a_hi, b_hi) + (f(a_lo, b_hi) + f(a_hi, b_lo))


def _dense_body(x_ref, pr_ref, oa_ref, oc_ref, we_ref, be_ref,
                w1_ref, b1_ref, w2_ref, b2_ref, eps_ref, wr_ref, br_ref,
                gamma_ref, beta_ref, out_ref):
    hp = jax.lax.Precision.HIGHEST
    nw = oa_ref.shape[0]
    ones_w = jnp.ones((nw, 1), jnp.float32)
    contract0 = (((0,), (0,)), ((), ()))
    asum = jax.lax.dot_general(oa_ref[...], ones_w, contract0,
                               precision=hp,
                               preferred_element_type=jnp.float32)
    cnt = jax.lax.dot_general(oc_ref[...], ones_w, contract0,
                              precision=hp,
                              preferred_element_type=jnp.float32)
    ps = pr_ref[0] + pr_ref[1]
    sums = ps + asum * we_ref[...] + cnt * be_ref[...]
    aggr = jnp.where(cnt > 0.0, sums / jnp.maximum(cnt, 1.0), 0.0)
    xb = x_ref[...]
    h = (1.0 + eps_ref[0, 0]) * xb + aggr
    t = jnp.maximum(_mm3(h, w1_ref[...]) + b1_ref[...], 0.0)
    o = _mm3(t, w2_ref[...]) + b2_ref[...]
    o = o + _mm3(xb, wr_ref[...]) + br_ref[...]
    o = o * (gamma_ref[...] * (1.0 / math.sqrt(1.0 + 1e-5))) + beta_ref[...]
    out_ref[...] = jnp.maximum(o, 0.0)


def _dense(x, pr, oa, oc, We, be, W1, b1, W2, b2, eps, Wr, br, gamma, beta,
           interpret=False):
    n, d = x.shape
    nc, n_pad, _ = pr.shape
    nw = oa.shape[0]
    oa2 = oa.reshape(nw, n_pad)
    oc2 = oc.reshape(nw, n_pad)
    blk = 2048
    nb = -(-n // blk)
    assert n_pad >= nb * blk

    full = lambda s: pl.BlockSpec(s, lambda i: (0,) * len(s))

    return pl.pallas_call(
        _dense_body,
        grid=(nb,),
        in_specs=[
            pl.BlockSpec((blk, d), lambda i: (i, 0)),
            pl.BlockSpec((nc, blk, d), lambda i: (0, i, 0)),
            pl.BlockSpec((nw, blk), lambda i: (0, i)),
            pl.BlockSpec((nw, blk), lambda i: (0, i)),
            full((1, d)),
            full((1, d)),
            full((d, d)),
            full((1, d)),
            full((d, d)),
            full((1, d)),
            full((1, 1)),
            full((d, d)),
            full((1, d)),
            full((1, d)),
            full((1, d)),
        ],
        out_specs=pl.BlockSpec((blk, d), lambda i: (i, 0)),
        out_shape=jax.ShapeDtypeStruct((n, d), jnp.float32),
        interpret=interpret,
    )(x, pr, oa2, oc2, We, be.reshape(1, d), W1, b1.reshape(1, d), W2,
      b2.reshape(1, d), eps.reshape(1, 1), Wr, br.reshape(1, d),
      gamma.reshape(1, d), beta.reshape(1, d))


def kernel(x, edge_index, edge_attr, We, be, W1, b1, W2, b2, eps, Wr, br,
           gamma, beta):
    src = edge_index[0]
    dst = edge_index[1]
    attr = edge_attr[:, 0]
    pr, oa, oc = _sc_aggregate(x, src, dst, attr)
    return _dense(x, pr, oa, oc, We, be, W1, b1, W2, b2, eps, Wr, br,
                  gamma, beta)


# speedup vs baseline: 12.0887x; 1.0002x over previous
import dataclasses
import functools
import math

import jax
import jax.numpy as jnp
from jax import lax
from jax.experimental import pallas as pl
from jax.experimental.pallas import tpu as pltpu
from jax.experimental.pallas import tpu_sc as plsc

NC = 2
NS = 16
L = 16
CHUNK = 80


def _sc_aggregate(x, src, dst, attr):
    n, d = x.shape
    e = src.shape[0]
    nw = NC * NS
    assert e % (nw * CHUNK) == 0
    e_per_w = e // nw
    n_chunks = e_per_w // CHUNK
    rps = -(-n // (NS * 128)) * 128
    n_pad = rps * NS
    ar = n_pad // 128

    zrows = jnp.zeros((rps, d), jnp.float32)
    zacc = jnp.zeros((ar, 128), jnp.float32)

    mesh = plsc.VectorSubcoreMesh(core_axis_name="c", subcore_axis_name="s")

    cp = pltpu.CompilerParams()
    if "needs_layout_passes" in pltpu.CompilerParams.__dataclass_fields__:
        cp = dataclasses.replace(cp, needs_layout_passes=False)

    assert n_chunks % 4 == 1 and n_chunks >= 9

    @functools.partial(
        pl.kernel,
        compiler_params=cp,
        out_type=[
            jax.ShapeDtypeStruct((NC, n_pad, d), jnp.float32),
            jax.ShapeDtypeStruct((nw, ar, 128), jnp.float32),
            jax.ShapeDtypeStruct((nw, ar, 128), jnp.float32),
        ],
        mesh=mesh,
        scratch_types=(
            [pltpu.VMEM((CHUNK,), jnp.int32)] * 4
            + [pltpu.VMEM((CHUNK,), jnp.int32)] * 4
            + [pltpu.VMEM((CHUNK,), jnp.float32)] * 4
            + [pltpu.VMEM((CHUNK, d), jnp.float32)] * 2
            + [pltpu.VMEM((ar, 128), jnp.float32)] * 2
            + [pltpu.VMEM_SHARED((n_pad, d), jnp.float32)]
            + [pltpu.SemaphoreType.DMA] * 8
        ),
    )
    def agg(x_hbm, src_hbm, dst_hbm, attr_hbm, zr_hbm, za_hbm,
            pr_hbm, oa_hbm, oc_hbm,
            si0, si1, si2, si3, di0, di1, di2, di3, at0, at1, at2, at3,
            rows0, rows1, asum, acnt, srows,
            semI0, semI1, semI2, semI3, semR0, semR1, semS0, semS1):
        cid = lax.axis_index("c")
        sid = lax.axis_index("s")
        wid = sid * NC + cid
        base0 = wid * e_per_w

        SI = [si0, si1, si2, si3]
        DI = [di0, di1, di2, di3]
        AT = [at0, at1, at2, at3]
        SEMI = [semI0, semI1, semI2, semI3]
        ROWS = [rows0, rows1]
        SEMR = [semR0, semR1]
        SEMS = [semS0, semS1]

        ones = jnp.ones((L,), jnp.float32)

        def issue_idx(c, k4):
            b = base0 + c * CHUNK
            pltpu.async_copy(src_hbm.at[pl.ds(b, CHUNK)], SI[k4], SEMI[k4])
            pltpu.async_copy(dst_hbm.at[pl.ds(b, CHUNK)], DI[k4], SEMI[k4])
            pltpu.async_copy(attr_hbm.at[pl.ds(b, CHUNK)], AT[k4], SEMI[k4])

        def wait_idx(c, k4):
            b = base0 + c * CHUNK
            pltpu.make_async_copy(src_hbm.at[pl.ds(b, CHUNK)], SI[k4],
                                  SEMI[k4]).wait()
            pltpu.make_async_copy(dst_hbm.at[pl.ds(b, CHUNK)], DI[k4],
                                  SEMI[k4]).wait()
            pltpu.make_async_copy(attr_hbm.at[pl.ds(b, CHUNK)], AT[k4],
                                  SEMI[k4]).wait()

        def issue_gather(k4, k2):
            pltpu.async_copy(x_hbm.at[SI[k4]], ROWS[k2], SEMR[k2])

        def wait_gather(k4, k2):
            pltpu.make_async_copy(x_hbm.at[SI[k4]], ROWS[k2],
                                  SEMR[k2]).wait()

        def aux_work(k4):
            for g in range(CHUNK // L):
                dvec = DI[k4][pl.ds(g * L, L)]
                avec = AT[k4][pl.ds(g * L, L)]
                plsc.addupdate_scatter(asum, [dvec >> 7, dvec & 127], avec)
                plsc.addupdate_scatter(acnt, [dvec >> 7, dvec & 127], ones)

        def scatter(k4, k2):
            pltpu.async_copy(ROWS[k2], srows.at[DI[k4]], SEMS[k2],
                             add=True)

        def wait_scatter(k4, k2):
            pltpu.make_async_copy(ROWS[k2], srows.at[DI[k4]],
                                  SEMS[k2]).wait()

        issue_idx(0, 0)
        issue_idx(1, 1)
        issue_idx(2, 2)
        issue_idx(3, 3)
        wait_idx(0, 0)
        issue_gather(0, 0)
        wait_idx(1, 1)
        issue_gather(1, 1)
        pltpu.sync_copy(zr_hbm, srows.at[pl.ds(sid * rps, rps)])
        pltpu.sync_copy(za_hbm, asum)
        pltpu.sync_copy(za_hbm, acnt)
        plsc.subcore_barrier()
        aux_work(0)
        aux_work(1)
        wait_gather(0, 0)
        scatter(0, 0)

        @pl.loop(0, (n_chunks - 5) // 4)
        def _(j):
            c_base = 4 * j + 2
            for o in range(4):
                c = c_base + o
                k4 = (2 + o) % 4
                k2 = o % 2
                wait_idx(c, k4)
                wait_scatter((k4 - 2) % 4, k2)
                issue_gather(k4, k2)
                aux_work(k4)
                wait_gather((k4 - 1) % 4, (k2 + 1) % 2)
                scatter((k4 - 1) % 4, (k2 + 1) % 2)
                issue_idx(c + 2, (k4 + 2) % 4)

        nc3 = n_chunks - 3
        wait_idx(nc3, 2)
        wait_scatter(0, 0)
        issue_gather(2, 0)
        aux_work(2)
        wait_gather(1, 1)
        scatter(1, 1)
        issue_idx(nc3 + 2, 0)
        wait_idx(nc3 + 1, 3)
        wait_scatter(1, 1)
        issue_gather(3, 1)
        aux_work(3)
        wait_gather(2, 0)
        scatter(2, 0)
        wait_idx(nc3 + 2, 0)
        wait_scatter(2, 0)
        issue_gather(0, 0)
        aux_work(0)
        wait_gather(3, 1)
        scatter(3, 1)
        wait_gather(0, 0)
        scatter(0, 0)
        wait_scatter(3, 1)
        wait_scatter(0, 0)

        pltpu.sync_copy(asum, oa_hbm.at[wid])
        pltpu.sync_copy(acnt, oc_hbm.at[wid])
        plsc.subcore_barrier()
        pltpu.sync_copy(srows.at[pl.ds(sid * rps, rps)],
                        pr_hbm.at[cid, pl.ds(sid * rps, rps)])

    return agg(x, src, dst, attr, zrows, zacc)


def _mm3(a, b):
    hp = jax.lax.Precision.DEFAULT
    a_hi = a.astype(jnp.bfloat16)
    a_lo = (a - a_hi.astype(jnp.float32)).astype(jnp.bfloat16)
    b_hi = b.astype(jnp.bfloat16)
    b_lo = (b - b_hi.astype(jnp.float32)).astype(jnp.bfloat16)
    f = lambda u, v: jnp.dot(u, v, preferred_element_type=jnp.float32,
                             precision=hp)
    return f(a_hi, b_hi) + (f(a_lo, b_hi) + f(a_hi, b_lo))


def _dense_body(x_ref, pr_ref, oa_ref, oc_ref, we_ref, be_ref,
                w1_ref, b1_ref, w2_ref, b2_ref, eps_ref, wr_ref, br_ref,
                gamma_ref, beta_ref, out_ref):
    hp = jax.lax.Precision.HIGHEST
    nw = oa_ref.shape[0]
    ones_w = jnp.ones((nw, 1), jnp.float32)
    contract0 = (((0,), (0,)), ((), ()))
    asum = jax.lax.dot_general(oa_ref[...], ones_w, contract0,
                               precision=hp,
                               preferred_element_type=jnp.float32)
    cnt = jax.lax.dot_general(oc_ref[...], ones_w, contract0,
                              precision=hp,
                              preferred_element_type=jnp.float32)
    ps = pr_ref[0] + pr_ref[1]
    sums = ps + asum * we_ref[...] + cnt * be_ref[...]
    aggr = jnp.where(cnt > 0.0, sums / jnp.maximum(cnt, 1.0), 0.0)
    xb = x_ref[...]
    h = (1.0 + eps_ref[0, 0]) * xb + aggr
    t = jnp.maximum(_mm3(h, w1_ref[...]) + b1_ref[...], 0.0)
    o = _mm3(t, w2_ref[...]) + b2_ref[...]
    o = o + _mm3(xb, wr_ref[...]) + br_ref[...]
    o = o * (gamma_ref[...] * (1.0 / math.sqrt(1.0 + 1e-5))) + beta_ref[...]
    out_ref[...] = jnp.maximum(o, 0.0)


def _dense(x, pr, oa, oc, We, be, W1, b1, W2, b2, eps, Wr, br, gamma, beta,
           interpret=False):
    n, d = x.shape
    nc, n_pad, _ = pr.shape
    nw = oa.shape[0]
    oa2 = oa.reshape(nw, n_pad)
    oc2 = oc.reshape(nw, n_pad)
    blk = 2048
    nb = -(-n // blk)
    assert n_pad >= nb * blk

    full = lambda s: pl.BlockSpec(s, lambda i: (0,) * len(s))

    return pl.pallas_call(
        _dense_body,
        grid=(nb,),
        in_specs=[
            pl.BlockSpec((blk, d), lambda i: (i, 0)),
            pl.BlockSpec((nc, blk, d), lambda i: (0, i, 0)),
            pl.BlockSpec((nw, blk), lambda i: (0, i)),
            pl.BlockSpec((nw, blk), lambda i: (0, i)),
            full((1, d)),
            full((1, d)),
            full((d, d)),
            full((1, d)),
            full((d, d)),
            full((1, d)),
            full((1, 1)),
            full((d, d)),
            full((1, d)),
            full((1, d)),
            full((1, d)),
        ],
        out_specs=pl.BlockSpec((blk, d), lambda i: (i, 0)),
        out_shape=jax.ShapeDtypeStruct((n, d), jnp.float32),
        interpret=interpret,
    )(x, pr, oa2, oc2, We, be.reshape(1, d), W1, b1.reshape(1, d), W2,
      b2.reshape(1, d), eps.reshape(1, 1), Wr, br.reshape(1, d),
      gamma.reshape(1, d), beta.reshape(1, d))


def kernel(x, edge_index, edge_attr, We, be, W1, b1, W2, b2, eps, Wr, br,
           gamma, beta):
    src = edge_index[0]
    dst = edge_index[1]
    attr = edge_attr[:, 0]
    pr, oa, oc = _sc_aggregate(x, src, dst, attr)
    return _dense(x, pr, oa, oc, We, be, W1, b1, W2, b2, eps, Wr, br,
                  gamma, beta)
